# baseline (speedup 1.0000x reference)
import functools

import jax
import jax.numpy as jnp
import numpy as np
from jax.experimental import pallas as pl
from jax.experimental.pallas import tpu as pltpu

D_MODEL = 1024
N_ROWS = 32768
ROW_BLK = 256
N_BLKS = N_ROWS // ROW_BLK

_div = np.exp(np.arange(0, D_MODEL, 2).astype(np.float32)
              * (-np.log(10000.0) / D_MODEL))
_TERM = np.repeat(_div, 2).reshape(1, D_MODEL)
_PHASE = ((np.arange(D_MODEL) % 2) * (np.pi / 2)).astype(np.float32).reshape(1, D_MODEL)


def _pe_rows_kernel(pos_ref, term_ref, phase_ref, out_ref):
    pos = pos_ref[0, 0, :].astype(jnp.float32)
    ang = pos[:, None] * term_ref[0, :][None, :] + phase_ref[0, :][None, :]
    out_ref[...] = jnp.sin(ang)


@jax.jit
def _pe_rows(pos3d, term, phase):
    return pl.pallas_call(
        _pe_rows_kernel,
        grid=(N_BLKS,),
        in_specs=[
            pl.BlockSpec((1, 1, ROW_BLK), lambda i: (i, 0, 0)),
            pl.BlockSpec((1, D_MODEL), lambda i: (0, 0)),
            pl.BlockSpec((1, D_MODEL), lambda i: (0, 0)),
        ],
        out_specs=pl.BlockSpec((ROW_BLK, D_MODEL), lambda i: (i, 0)),
        out_shape=jax.ShapeDtypeStruct((N_ROWS, D_MODEL), jnp.float32),
    )(pos3d, term, phase)


def kernel(positions, pe):
    pos3d = positions.reshape(N_BLKS, 1, ROW_BLK).astype(jnp.int32)
    out = _pe_rows(pos3d, jnp.asarray(_TERM), jnp.asarray(_PHASE))
    return out.reshape(positions.shape[0], positions.shape[1], D_MODEL)


# speedup vs baseline: 3.7961x; 3.7961x over previous
import functools

import jax
import jax.numpy as jnp
from jax import lax
from jax.experimental import pallas as pl
from jax.experimental.pallas import tpu as pltpu
from jax.experimental.pallas import tpu_sc as plsc

D_MODEL = 1024
N_ROWS = 32768
NC, NS = 2, 16
NW = NC * NS
ROWS_PER_W = N_ROWS // NW
CHUNK = 32
N_CHUNKS = ROWS_PER_W // CHUNK


def _make_gather():
    mesh = plsc.VectorSubcoreMesh(
        core_axis_name="c", subcore_axis_name="s",
        num_cores=NC, num_subcores=NS)

    @functools.partial(
        pl.kernel,
        out_type=jax.ShapeDtypeStruct((N_ROWS, D_MODEL), jnp.float32),
        mesh=mesh,
        scratch_types=[
            pltpu.VMEM((N_CHUNKS, CHUNK), jnp.int32),
            pltpu.VMEM((CHUNK, D_MODEL), jnp.float32),
            pltpu.VMEM((CHUNK, D_MODEL), jnp.float32),
            pltpu.SemaphoreType.DMA,
            pltpu.SemaphoreType.DMA,
            pltpu.SemaphoreType.DMA,
            pltpu.SemaphoreType.DMA,
        ],
    )
    def gather_kernel(idx_hbm, table_hbm, out_hbm, idx_v, buf0, buf1,
                      sem0, sem1, wsem0, wsem1):
        wid = lax.axis_index("s") * NC + lax.axis_index("c")
        base = wid * ROWS_PER_W
        pltpu.sync_copy(idx_hbm.at[wid], idx_v)

        def start_gather(j, buf, sem):
            pltpu.make_async_copy(table_hbm.at[idx_v.at[j]], buf, sem).start()

        def wait_gather(j, buf, sem):
            pltpu.make_async_copy(table_hbm.at[idx_v.at[j]], buf, sem).wait()

        def start_write(j, buf, sem):
            pltpu.make_async_copy(
                buf, out_hbm.at[pl.ds(base + j * CHUNK, CHUNK)], sem).start()

        def wait_write(j, buf, sem):
            pltpu.make_async_copy(
                buf, out_hbm.at[pl.ds(base + j * CHUNK, CHUNK)], sem).wait()

        start_gather(0, buf0, sem0)
        start_gather(1, buf1, sem1)

        def body(t, _):
            j0 = 2 * t
            j1 = j0 + 1
            wait_gather(j0, buf0, sem0)
            start_write(j0, buf0, wsem0)
            wait_gather(j1, buf1, sem1)
            start_write(j1, buf1, wsem1)

            wait_write(j0, buf0, wsem0)

            @pl.when(j0 + 2 < N_CHUNKS)
            def _():
                start_gather(j0 + 2, buf0, sem0)

            wait_write(j1, buf1, wsem1)

            @pl.when(j1 + 2 < N_CHUNKS)
            def _():
                start_gather(j1 + 2, buf1, sem1)

            return ()

        lax.fori_loop(0, N_CHUNKS // 2, body, (), unroll=False)

    return gather_kernel


_gather = _make_gather()


def kernel(positions, pe):
    idx = positions.reshape(NW, N_CHUNKS, CHUNK).astype(jnp.int32)
    out = _gather(idx, pe)
    return out.reshape(positions.shape[0], positions.shape[1], D_MODEL)


# speedup vs baseline: 3.8029x; 1.0018x over previous
import functools

import jax
import jax.numpy as jnp
from jax import lax
from jax.experimental import pallas as pl
from jax.experimental.pallas import tpu as pltpu
from jax.experimental.pallas import tpu_sc as plsc

D_MODEL = 1024
N_ROWS = 32768
NC, NS = 2, 16
NW = NC * NS
ROWS_PER_W = N_ROWS // NW
CHUNK = 32
NBUF = 3
N_CHUNKS = ROWS_PER_W // CHUNK
N_FULL = (N_CHUNKS // NBUF) * NBUF
TAIL = N_CHUNKS - N_FULL


def _make_gather():
    mesh = plsc.VectorSubcoreMesh(
        core_axis_name="c", subcore_axis_name="s",
        num_cores=NC, num_subcores=NS)

    @functools.partial(
        pl.kernel,
        out_type=jax.ShapeDtypeStruct((N_ROWS, D_MODEL), jnp.float32),
        mesh=mesh,
        scratch_types=(
            [pltpu.VMEM((N_CHUNKS, CHUNK), jnp.int32)]
            + [pltpu.VMEM((CHUNK, D_MODEL), jnp.float32)] * NBUF
            + [pltpu.SemaphoreType.DMA] * (2 * NBUF)
        ),
    )
    def gather_kernel(idx_hbm, table_hbm, out_hbm, idx_v, *bufs_and_sems):
        bufs = bufs_and_sems[:NBUF]
        gsems = bufs_and_sems[NBUF:2 * NBUF]
        wsems = bufs_and_sems[2 * NBUF:]
        wid = lax.axis_index("s") * NC + lax.axis_index("c")
        base = wid * ROWS_PER_W
        pltpu.sync_copy(idx_hbm.at[wid], idx_v)

        def start_gather(j, b):
            pltpu.make_async_copy(
                table_hbm.at[idx_v.at[j]], bufs[b], gsems[b]).start()

        def wait_gather(j, b):
            pltpu.make_async_copy(
                table_hbm.at[idx_v.at[j]], bufs[b], gsems[b]).wait()

        def start_write(j, b):
            pltpu.make_async_copy(
                bufs[b], out_hbm.at[pl.ds(base + j * CHUNK, CHUNK)],
                wsems[b]).start()

        def wait_write(j, b):
            pltpu.make_async_copy(
                bufs[b], out_hbm.at[pl.ds(base + j * CHUNK, CHUNK)],
                wsems[b]).wait()

        for b in range(NBUF):
            start_gather(b, b)

        def body(t, _):
            for b in range(NBUF):
                j = NBUF * t + b
                wait_gather(j, b)
                start_write(j, b)
            for b in range(NBUF):
                j = NBUF * t + b
                wait_write(j, b)

                @pl.when(j + NBUF < N_CHUNKS)
                def _():
                    start_gather(j + NBUF, b)

            return ()

        lax.fori_loop(0, N_FULL // NBUF, body, (), unroll=False)

        for b in range(TAIL):
            j = N_FULL + b
            wait_gather(j, b)
            start_write(j, b)
        for b in range(TAIL):
            wait_write(N_FULL + b, b)

    return gather_kernel


_gather = _make_gather()


def kernel(positions, pe):
    idx = positions.reshape(NW, N_CHUNKS, CHUNK).astype(jnp.int32)
    out = _gather(idx, pe)
    return out.reshape(positions.shape[0], positions.shape[1], D_MODEL)


# speedup vs baseline: 3.9458x; 1.0376x over previous
import functools

import jax
import jax.numpy as jnp
from jax import lax
from jax.experimental import pallas as pl
from jax.experimental.pallas import tpu as pltpu
from jax.experimental.pallas import tpu_sc as plsc

D_MODEL = 1024
N_ROWS = 32768
NC, NS = 2, 16
NW = NC * NS
ROWS_PER_W = N_ROWS // NW
CHUNK_A = 64
CHUNK_B = 32
PAIR = CHUNK_A + CHUNK_B
N_PAIRS = 10
TAIL = ROWS_PER_W - N_PAIRS * PAIR


def _make_gather():
    mesh = plsc.VectorSubcoreMesh(
        core_axis_name="c", subcore_axis_name="s",
        num_cores=NC, num_subcores=NS)

    @functools.partial(
        pl.kernel,
        out_type=jax.ShapeDtypeStruct((N_ROWS, D_MODEL), jnp.float32),
        mesh=mesh,
        scratch_types=[
            pltpu.VMEM((ROWS_PER_W,), jnp.int32),
            pltpu.VMEM((CHUNK_A, D_MODEL), jnp.float32),
            pltpu.VMEM((CHUNK_B, D_MODEL), jnp.float32),
            pltpu.SemaphoreType.DMA,
            pltpu.SemaphoreType.DMA,
        ],
    )
    def gather_kernel(idx_hbm, table_hbm, out_hbm, idx_v, buf_a, buf_b,
                      sem_a, sem_b):
        wid = lax.axis_index("s") * NC + lax.axis_index("c")
        base = wid * ROWS_PER_W
        pltpu.sync_copy(idx_hbm.at[wid], idx_v)

        def start_a(r):
            pltpu.make_async_copy(
                table_hbm.at[idx_v.at[pl.ds(r, CHUNK_A)]], buf_a, sem_a).start()

        def wait_a(r):
            pltpu.make_async_copy(
                table_hbm.at[idx_v.at[pl.ds(r, CHUNK_A)]], buf_a, sem_a).wait()

        def start_b(r):
            pltpu.make_async_copy(
                table_hbm.at[idx_v.at[pl.ds(r, CHUNK_B)]], buf_b, sem_b).start()

        def wait_b(r):
            pltpu.make_async_copy(
                table_hbm.at[idx_v.at[pl.ds(r, CHUNK_B)]], buf_b, sem_b).wait()

        def write_a(r):
            pltpu.sync_copy(buf_a, out_hbm.at[pl.ds(base + r, CHUNK_A)])

        def write_b(r):
            pltpu.sync_copy(buf_b, out_hbm.at[pl.ds(base + r, CHUNK_B)])

        start_a(0)
        start_b(CHUNK_A)

        def body(t, _):
            r = t * PAIR
            wait_a(r)
            write_a(r)
            start_a(r + PAIR)
            wait_b(r + CHUNK_A)
            write_b(r + CHUNK_A)

            @pl.when(t + 1 < N_PAIRS)
            def _():
                start_b(r + PAIR + CHUNK_A)

            return ()

        lax.fori_loop(0, N_PAIRS, body, (), unroll=False)

        wait_a(N_PAIRS * PAIR)
        write_a(N_PAIRS * PAIR)

    return gather_kernel


_gather = _make_gather()


def kernel(positions, pe):
    idx = positions.reshape(NW, ROWS_PER_W).astype(jnp.int32)
    out = _gather(idx, pe)
    return out.reshape(positions.shape[0], positions.shape[1], D_MODEL)


# speedup vs baseline: 4.0521x; 1.0269x over previous
import functools

import jax
import jax.numpy as jnp
from jax import lax
from jax.experimental import pallas as pl
from jax.experimental.pallas import tpu as pltpu
from jax.experimental.pallas import tpu_sc as plsc

D_MODEL = 1024
N_ROWS = 32768
NC, NS = 2, 16
NW = NC * NS
ROWS_PER_W = N_ROWS // NW
CHUNK = 32
NBUF = 3
N_CHUNKS = ROWS_PER_W // CHUNK
N_FULL = (N_CHUNKS // NBUF) * NBUF
TAIL = N_CHUNKS - N_FULL


def _make_gather():
    mesh = plsc.VectorSubcoreMesh(
        core_axis_name="c", subcore_axis_name="s",
        num_cores=NC, num_subcores=NS)

    @functools.partial(
        pl.kernel,
        out_type=jax.ShapeDtypeStruct((N_ROWS, D_MODEL), jnp.float32),
        mesh=mesh,
        scratch_types=(
            [pltpu.VMEM((N_CHUNKS, CHUNK), jnp.int32)]
            + [pltpu.VMEM((CHUNK, D_MODEL), jnp.float32)] * NBUF
            + [pltpu.SemaphoreType.DMA] * NBUF
        ),
    )
    def gather_kernel(idx_hbm, table_hbm, out_hbm, idx_v, *bufs_and_sems):
        bufs = bufs_and_sems[:NBUF]
        gsems = bufs_and_sems[NBUF:]
        wid = lax.axis_index("s") * NC + lax.axis_index("c")
        base = wid * ROWS_PER_W
        pltpu.sync_copy(idx_hbm.at[wid], idx_v)

        def start_gather(j, b):
            pltpu.make_async_copy(
                table_hbm.at[idx_v.at[j]], bufs[b], gsems[b]).start()

        def wait_gather(j, b):
            pltpu.make_async_copy(
                table_hbm.at[idx_v.at[j]], bufs[b], gsems[b]).wait()

        def write_out(j, b):
            pltpu.sync_copy(bufs[b], out_hbm.at[pl.ds(base + j * CHUNK, CHUNK)])

        for b in range(NBUF):
            start_gather(b, b)

        def body(t, _):
            for b in range(NBUF):
                j = NBUF * t + b
                wait_gather(j, b)
                write_out(j, b)

                @pl.when(j + NBUF < N_CHUNKS)
                def _():
                    start_gather(j + NBUF, b)

            return ()

        lax.fori_loop(0, N_FULL // NBUF, body, (), unroll=False)

        for b in range(TAIL):
            j = N_FULL + b
            wait_gather(j, b)
            write_out(j, b)

    return gather_kernel


_gather = _make_gather()


def kernel(positions, pe):
    idx = positions.reshape(NW, N_CHUNKS, CHUNK).astype(jnp.int32)
    out = _gather(idx, pe)
    return out.reshape(positions.shape[0], positions.shape[1], D_MODEL)
